# baseline (speedup 1.0000x reference)
import jax
import jax.numpy as jnp
from jax.experimental import pallas as pl
from jax.experimental.pallas import tpu as pltpu

B, N, D, C = 8, 1024, 128, 128
H = N // 2
SUB = 2
BS = H // SUB


def _body(a0_ref, a1_ref, x_ref, w_ref, b_ref, o_ref, xwb_ref):
    xwb_ref[...] = (
        jnp.dot(x_ref[0], w_ref[...], preferred_element_type=jnp.float32)
        + b_ref[...]
    )
    xwb = xwb_ref[...]
    for half, a_ref in enumerate((a0_ref, a1_ref)):
        for t in range(SUB):
            a = a_ref[0, t * BS:(t + 1) * BS, :]
            u = jnp.dot(a, xwb, preferred_element_type=jnp.float32)
            ss = jnp.sum(u * u, axis=1, keepdims=True)
            out = u * jax.lax.rsqrt(jnp.maximum(ss, 1e-24))
            s = jnp.maximum(out, 0.0)
            e = jnp.exp(s)
            row0 = half * H + t * BS
            o_ref[0, row0:row0 + BS, :] = e / jnp.sum(e, axis=1, keepdims=True)


@jax.jit
def kernel(input_tensor, tilda_adjacency_matrix, W, b):
    bias = b.reshape(1, C)
    return pl.pallas_call(
        _body,
        grid=(B,),
        in_specs=[
            pl.BlockSpec((1, H, N), lambda bi: (bi, 0, 0)),
            pl.BlockSpec((1, H, N), lambda bi: (bi, 1, 0)),
            pl.BlockSpec((1, N, D), lambda bi: (bi, 0, 0)),
            pl.BlockSpec((D, C), lambda bi: (0, 0)),
            pl.BlockSpec((1, C), lambda bi: (0, 0)),
        ],
        out_specs=pl.BlockSpec((1, N, C), lambda bi: (bi, 0, 0)),
        out_shape=jax.ShapeDtypeStruct((B, N, C), jnp.float32),
        scratch_shapes=[pltpu.VMEM((N, C), jnp.float32)],
        compiler_params=pltpu.CompilerParams(
            dimension_semantics=("arbitrary",),
        ),
    )(tilda_adjacency_matrix, tilda_adjacency_matrix, input_tensor, W, bias)


# speedup vs baseline: 1.1015x; 1.1015x over previous
import jax
import jax.numpy as jnp
from jax.experimental import pallas as pl
from jax.experimental.pallas import tpu as pltpu

B, N, D, C = 8, 1024, 128, 128
BN = 1024
PB = 2
SUB = 4
BS = BN // SUB


def _body(a_ref, x_ref, w_ref, b_ref, o_ref, xwb_ref):
    for g in range(PB):
        xwb_ref[g] = (
            jnp.dot(x_ref[g], w_ref[...], preferred_element_type=jnp.float32)
            + b_ref[...]
        )
    for g in range(PB):
        xwb = xwb_ref[g]
        for t in range(SUB):
            a = a_ref[g, t * BS:(t + 1) * BS, :]
            u = jnp.dot(a, xwb, preferred_element_type=jnp.float32)
            ss = jnp.sum(u * u, axis=1, keepdims=True)
            out = u * jax.lax.rsqrt(jnp.maximum(ss, 1e-24))
            s = jnp.maximum(out, 0.0)
            e = jnp.exp(s)
            o_ref[g, t * BS:(t + 1) * BS, :] = e / jnp.sum(e, axis=1, keepdims=True)


@jax.jit
def kernel(input_tensor, tilda_adjacency_matrix, W, b):
    bias = b.reshape(1, C)
    grid = (B // PB,)
    return pl.pallas_call(
        _body,
        grid=grid,
        in_specs=[
            pl.BlockSpec((PB, BN, N), lambda bi: (bi, 0, 0)),
            pl.BlockSpec((PB, N, D), lambda bi: (bi, 0, 0)),
            pl.BlockSpec((D, C), lambda bi: (0, 0)),
            pl.BlockSpec((1, C), lambda bi: (0, 0)),
        ],
        out_specs=pl.BlockSpec((PB, BN, C), lambda bi: (bi, 0, 0)),
        out_shape=jax.ShapeDtypeStruct((B, N, C), jnp.float32),
        scratch_shapes=[pltpu.VMEM((PB, N, C), jnp.float32)],
        compiler_params=pltpu.CompilerParams(
            dimension_semantics=("arbitrary",),
        ),
    )(tilda_adjacency_matrix, input_tensor, W, bias)
